# baseline (speedup 1.0000x reference)
import functools

import jax
import jax.numpy as jnp
from jax import lax
from jax.experimental import pallas as pl
from jax.experimental.pallas import tpu as pltpu, tpu_sc as plsc

N_ATOMS = 100000
EMB = 128
NROWS = 9
N_PAIRS = N_ATOMS // 2

_info = plsc.get_sparse_core_info()
_NC, _NS = _info.num_cores, _info.num_subcores
_NW = _NC * _NS

_CH = 112
_QW = 1568
_Q_LAST = N_PAIRS - (_NW - 1) * _QW
_NFULL = _QW // _CH
_NFULL_LAST = 11
_T_LAST = 80
_IN_PAD = _QW * _NW
_NBUF = 4

_mesh = plsc.VectorSubcoreMesh(core_axis_name="c", subcore_axis_name="s")


@functools.partial(
    pl.kernel,
    mesh=_mesh,
    out_type=jax.ShapeDtypeStruct((N_PAIRS, 2, EMB), jnp.float32),
    scratch_types=[
        pltpu.VMEM_SHARED((NROWS * NROWS, 2, EMB), jnp.float32),
        pltpu.VMEM((_QW,), jnp.int32),
        [pltpu.VMEM((_CH, 2, EMB), jnp.float32) for _ in range(_NBUF)],
        [pltpu.SemaphoreType.DMA for _ in range(_NBUF)],
        [pltpu.SemaphoreType.DMA for _ in range(_NBUF)],
    ],
)
def _gather_kernel(ptab_hbm, idx_hbm, out_hbm, ptab_sh, idx_v, bufs,
                   gsems, wsems):
    sid = lax.axis_index("s")
    wid = sid * _NC + lax.axis_index("c")
    base = wid * _QW

    @pl.when(sid == 0)
    def _():
        pltpu.sync_copy(ptab_hbm, ptab_sh)

    pltpu.sync_copy(idx_hbm.at[pl.ds(base, _QW)], idx_v)
    plsc.subcore_barrier()

    def gather(k, b):
        return pltpu.async_copy(
            ptab_sh.at[idx_v.at[pl.ds(k * _CH, _CH)]], bufs[b], gsems[b])

    def run_chunks(n_chunks):
        gathers = [None] * n_chunks
        writes = [None] * n_chunks
        for j in range(min(_NBUF - 1, n_chunks)):
            gathers[j] = gather(j, j)
        for k in range(n_chunks):
            b = k % _NBUF
            pre = k + _NBUF - 1
            if pre < n_chunks:
                if k > 0:
                    writes[k - 1].wait()
                gathers[pre] = gather(pre, pre % _NBUF)
            gathers[k].wait()
            writes[k] = pltpu.async_copy(
                bufs[b], out_hbm.at[pl.ds(base + k * _CH, _CH)], wsems[b])
        for k in range(max(0, n_chunks - _NBUF), n_chunks):
            writes[k].wait()

    @pl.when(wid < _NW - 1)
    def _():
        run_chunks(_NFULL)

    @pl.when(wid == _NW - 1)
    def _():
        run_chunks(_NFULL_LAST)
        for t in range(2):
            tail_off = _NFULL_LAST * _CH + t * _T_LAST
            tb = bufs[t].at[pl.ds(0, _T_LAST)]
            pltpu.async_copy(
                ptab_sh.at[idx_v.at[pl.ds(tail_off, _T_LAST)]],
                tb, gsems[t]).wait()
            pltpu.sync_copy(tb, out_hbm.at[pl.ds(base + tail_off, _T_LAST)])


def kernel(C, table):
    t = table.astype(jnp.float32)
    ptab = jnp.stack(
        [jnp.repeat(t, NROWS, axis=0), jnp.tile(t, (NROWS, 1))], axis=1)
    c = C.astype(jnp.int32).reshape(N_PAIRS, 2)
    pidx = c[:, 0] * NROWS + c[:, 1]
    pidx = jnp.pad(pidx, (0, _IN_PAD - N_PAIRS))
    out = _gather_kernel(ptab, pidx)
    return out.reshape(N_ATOMS, EMB)


# speedup vs baseline: 1.2404x; 1.2404x over previous
import functools

import jax
import jax.numpy as jnp
from jax import lax
from jax.experimental import pallas as pl
from jax.experimental.pallas import tpu as pltpu, tpu_sc as plsc

N_ATOMS = 100000
EMB = 128
NROWS = 9

_info = plsc.get_sparse_core_info()
_NC, _NS = _info.num_cores, _info.num_subcores
_NW = _NC * _NS

_CH = 112
_QW = 3128
_Q_LAST = N_ATOMS - (_NW - 1) * _QW
_NV = 8
_VROWS = _NV * _CH
_NS_CH = (_QW - _VROWS) // _CH
_T_MAIN = _QW - _VROWS - _NS_CH * _CH
_T_LAST = _Q_LAST - _VROWS - _NS_CH * _CH
_IN_PAD = _QW * _NW
_NBUF = 4
_NVBUF = 2

_mesh = plsc.VectorSubcoreMesh(core_axis_name="c", subcore_axis_name="s")


@functools.partial(
    pl.kernel,
    mesh=_mesh,
    compiler_params=pltpu.CompilerParams(needs_layout_passes=False),
    out_type=jax.ShapeDtypeStruct((N_ATOMS, EMB), jnp.float32),
    scratch_types=[
        pltpu.VMEM_SHARED((NROWS, EMB), jnp.float32),
        pltpu.VMEM((NROWS, EMB), jnp.float32),
        pltpu.VMEM((_QW,), jnp.int32),
        [pltpu.VMEM((_CH, EMB), jnp.float32) for _ in range(_NBUF)],
        [pltpu.VMEM((_CH, EMB), jnp.float32) for _ in range(_NVBUF)],
        [pltpu.SemaphoreType.DMA for _ in range(_NBUF)],
        [pltpu.SemaphoreType.DMA for _ in range(_NBUF)],
        [pltpu.SemaphoreType.DMA for _ in range(_NVBUF)],
    ],
)
def _emb_kernel(table_hbm, idx_hbm, out_hbm, tab_sh, tab_2d, idx_v,
                sbufs, vbufs, gsems, wsems, vsems):
    sid = lax.axis_index("s")
    wid = sid * _NC + lax.axis_index("c")
    base = wid * _QW
    s0 = _VROWS

    @pl.when(sid == 0)
    def _():
        pltpu.sync_copy(table_hbm, tab_sh)

    pltpu.sync_copy(table_hbm, tab_2d)
    pltpu.sync_copy(idx_hbm.at[pl.ds(base, _QW)], idx_v)
    plsc.subcore_barrier()

    lane = lax.broadcasted_iota(jnp.int32, (16,), 0)
    col16 = tuple(16 * j + lane for j in range(EMB // 16))

    def vchunk(vk, buf):
        def group(g, carry):
            cvec = idx_v[pl.ds(vk * _CH + g * 16, 16)]
            for a in range(16):
                sel = jnp.full((16,), a, jnp.int32)
                bcrow = cvec.at[sel].get(mode="promise_in_bounds")
                r = g * 16 + a
                for j in range(EMB // 16):
                    vals = plsc.load_gather(tab_2d, [bcrow, col16[j]])
                    buf[r, pl.ds(16 * j, 16)] = vals
            return carry
        lax.fori_loop(0, _CH // 16, group, 0)

    def sgather(k, b):
        return pltpu.async_copy(
            tab_sh.at[idx_v.at[pl.ds(s0 + k * _CH, _CH)]], sbufs[b],
            gsems[b])

    def vflush(vk, vb):
        return pltpu.async_copy(
            vbufs[vb], out_hbm.at[pl.ds(base + vk * _CH, _CH)], vsems[vb])

    sg = [None] * _NS_CH
    sw = [None] * _NS_CH
    vw = [None] * _NV
    for j in range(_NBUF - 1):
        sg[j] = sgather(j, j)

    vk = 0
    for k in range(_NS_CH):
        pre = k + _NBUF - 1
        if pre < _NS_CH:
            if k > 0:
                sw[k - 1].wait()
            sg[pre] = sgather(pre, pre % _NBUF)
        if k % 2 == 0 and vk < _NV:
            vb = vk % _NVBUF
            if vk >= _NVBUF:
                vw[vk - _NVBUF].wait()
            vchunk(vk, vbufs[vb])
            vw[vk] = vflush(vk, vb)
            vk += 1
        sg[k].wait()
        sw[k] = pltpu.async_copy(
            sbufs[k % _NBUF], out_hbm.at[pl.ds(base + s0 + k * _CH, _CH)],
            wsems[k % _NBUF])

    for k in range(_NS_CH - _NBUF, _NS_CH):
        sw[k].wait()

    @pl.when(wid < _NW - 1)
    def _():
        tb = sbufs[_NS_CH % _NBUF].at[pl.ds(0, _T_MAIN)]
        pltpu.async_copy(
            tab_sh.at[idx_v.at[pl.ds(s0 + _NS_CH * _CH, _T_MAIN)]],
            tb, gsems[_NS_CH % _NBUF]).wait()
        pltpu.sync_copy(
            tb, out_hbm.at[pl.ds(base + s0 + _NS_CH * _CH, _T_MAIN)])

    @pl.when(wid == _NW - 1)
    def _():
        tb = sbufs[_NS_CH % _NBUF].at[pl.ds(0, _T_LAST)]
        pltpu.async_copy(
            tab_sh.at[idx_v.at[pl.ds(s0 + _NS_CH * _CH, _T_LAST)]],
            tb, gsems[_NS_CH % _NBUF]).wait()
        pltpu.sync_copy(
            tb, out_hbm.at[pl.ds(base + s0 + _NS_CH * _CH, _T_LAST)])

    while vk < _NV:
        vb = vk % _NVBUF
        if vk >= _NVBUF:
            vw[vk - _NVBUF].wait()
        vchunk(vk, vbufs[vb])
        vw[vk] = vflush(vk, vb)
        vk += 1
    for v in range(_NV - _NVBUF, _NV):
        vw[v].wait()


def kernel(C, table):
    idx = jnp.pad(C.astype(jnp.int32), (0, _IN_PAD - N_ATOMS))
    return _emb_kernel(table.astype(jnp.float32), idx)


# speedup vs baseline: 2.0515x; 1.6539x over previous
import functools

import jax
import jax.numpy as jnp
from jax import lax
from jax.experimental import pallas as pl
from jax.experimental.pallas import tpu as pltpu, tpu_sc as plsc

N_ATOMS = 100000
EMB = 128
NROWS = 9

_info = plsc.get_sparse_core_info()
_NC, _NS = _info.num_cores, _info.num_subcores
_NW = _NC * _NS

_CH = 112
_QW = 3128
_Q_LAST = N_ATOMS - (_NW - 1) * _QW
_NFULL = _Q_LAST // _CH
_T_MAIN = _QW - _NFULL * _CH
_T_LAST = _Q_LAST - _NFULL * _CH
_NBUF = 4

_mesh = plsc.VectorSubcoreMesh(core_axis_name="c", subcore_axis_name="s")


@functools.partial(
    pl.kernel,
    mesh=_mesh,
    out_type=jax.ShapeDtypeStruct((N_ATOMS, EMB), jnp.float32),
    scratch_types=[
        pltpu.VMEM_SHARED((NROWS, EMB), jnp.float32),
        pltpu.VMEM((_QW,), jnp.int32),
        [pltpu.VMEM((_CH, EMB), jnp.float32) for _ in range(_NBUF)],
        [pltpu.SemaphoreType.DMA for _ in range(_NBUF)],
        [pltpu.SemaphoreType.DMA for _ in range(_NBUF)],
    ],
)
def _gather_kernel(table_hbm, idx_hbm, out_hbm, tab_sh, idx_v, bufs,
                   gsems, wsems):
    sid = lax.axis_index("s")
    wid = sid * _NC + lax.axis_index("c")
    base = wid * _QW

    @pl.when(sid == 0)
    def _():
        pltpu.sync_copy(table_hbm, tab_sh)

    @pl.when(wid < _NW - 1)
    def _():
        pltpu.sync_copy(idx_hbm.at[pl.ds(base, _QW)], idx_v)

    @pl.when(wid == _NW - 1)
    def _():
        pltpu.sync_copy(idx_hbm.at[pl.ds(base, _Q_LAST)],
                        idx_v.at[pl.ds(0, _Q_LAST)])

    plsc.subcore_barrier()

    def gather(k, b):
        return pltpu.async_copy(
            tab_sh.at[idx_v.at[pl.ds(k * _CH, _CH)]], bufs[b], gsems[b])

    gathers = [None] * _NFULL
    writes = [None] * _NFULL
    for j in range(_NBUF - 1):
        gathers[j] = gather(j, j)
    for k in range(_NFULL):
        b = k % _NBUF
        pre = k + _NBUF - 1
        if pre < _NFULL:
            if k > 0:
                writes[k - 1].wait()
            gathers[pre] = gather(pre, pre % _NBUF)
        gathers[k].wait()
        writes[k] = pltpu.async_copy(
            bufs[b], out_hbm.at[pl.ds(base + k * _CH, _CH)], wsems[b])
    for k in range(_NFULL - _NBUF, _NFULL):
        writes[k].wait()

    tail_off = _NFULL * _CH

    @pl.when(wid < _NW - 1)
    def _():
        tb = bufs[0].at[pl.ds(0, _T_MAIN)]
        pltpu.async_copy(
            tab_sh.at[idx_v.at[pl.ds(tail_off, _T_MAIN)]],
            tb, gsems[0]).wait()
        pltpu.sync_copy(tb, out_hbm.at[pl.ds(base + tail_off, _T_MAIN)])

    @pl.when(wid == _NW - 1)
    def _():
        tb = bufs[1].at[pl.ds(0, _T_LAST)]
        pltpu.async_copy(
            tab_sh.at[idx_v.at[pl.ds(tail_off, _T_LAST)]],
            tb, gsems[1]).wait()
        pltpu.sync_copy(tb, out_hbm.at[pl.ds(base + tail_off, _T_LAST)])


def kernel(C, table):
    return _gather_kernel(table.astype(jnp.float32), C.astype(jnp.int32))
